# baseline (speedup 1.0000x reference)
import functools

import jax
import jax.numpy as jnp
from jax import lax
from jax.experimental import pallas as pl
from jax.experimental.pallas import tpu as pltpu
from jax.experimental.pallas import tpu_sc as plsc

D_MODEL = 64
SCALE = 8.0
NC, NS, L = 2, 16, 16
NW = NC * NS
CH = 128
NBUF = 4


def _make_x_tc_relayout(seq: int, n_rows: int):
    tq, cq = seq // 8, n_rows // CH

    def body(xt_ref, out_ref):
        out_ref[...] = xt_ref[...].reshape(1, 1, 8, CH)

    return pl.pallas_call(
        body,
        grid=(tq, cq),
        in_specs=[pl.BlockSpec((8, CH), lambda i, j: (i, j))],
        out_specs=pl.BlockSpec((1, 1, 8, CH), lambda i, j: (i, j, 0, 0)),
        out_shape=jax.ShapeDtypeStruct((tq, cq, 8, CH), jnp.int32),
    )


def _make_sc_lookup(seq: int, n_rows: int):
    mesh = plsc.VectorSubcoreMesh(core_axis_name="c", subcore_axis_name="s")
    n_groups = seq // NBUF
    tq = seq // 8

    @functools.partial(
        pl.kernel,
        out_type=jax.ShapeDtypeStruct((seq, n_rows, D_MODEL), jnp.float32),
        mesh=mesh,
        scratch_types=[
            pltpu.VMEM((tq, 8, CH), jnp.int32),
            [pltpu.VMEM((CH, D_MODEL), jnp.float32)] * NBUF,
            [pltpu.VMEM((CH, D_MODEL), jnp.float32)] * NBUF,
            [pltpu.SemaphoreType.DMA] * NBUF,
        ],
        compiler_params=pltpu.CompilerParams(use_tc_tiling_on_sc=False),
    )
    def k(xq_hbm, table_hbm, out_hbm, idx_v, bufs, obufs, gsems):
        wid = lax.axis_index("s") * NC + lax.axis_index("c")
        col = wid * CH
        pltpu.sync_copy(xq_hbm.at[:, wid], idx_v)

        def g_start(t, b):
            pltpu.async_copy(
                table_hbm.at[idx_v.at[t // 8, t % 8]], bufs[b], gsems[b]
            )

        def g_wait(t, b):
            pltpu.make_async_copy(
                table_hbm.at[idx_v.at[t // 8, t % 8]], bufs[b], gsems[b]
            ).wait()

        def scale(b):
            buf, obuf = bufs[b], obufs[b]

            def srow(r, c2):
                for u in range(2):
                    for c in range(D_MODEL // L):
                        sl = pl.ds(c * L, L)
                        obuf[2 * r + u, sl] = buf[2 * r + u, sl] * SCALE
                return c2

            lax.fori_loop(0, CH // 2, srow, 0)

        def s_sync(t, b):
            pltpu.sync_copy(obufs[b], out_hbm.at[t, pl.ds(col, CH)])

        for b in range(NBUF):
            g_start(b, b)

        def step(g, carry):
            for b in range(NBUF):
                t = g * NBUF + b
                g_wait(t, b)
                scale(b)
                g_start(t + NBUF, b)
                s_sync(t, b)
            return carry

        lax.fori_loop(0, n_groups - 1, step, 0)

        for b in range(NBUF):
            t = (n_groups - 1) * NBUF + b
            g_wait(t, b)
            scale(b)
            s_sync(t, b)

    return k


def kernel(x, table):
    n_rows, seq = x.shape
    xq = _make_x_tc_relayout(seq, n_rows)(x.astype(jnp.int32).T)
    outp = _make_sc_lookup(seq, n_rows)(xq, table)
    return outp.transpose(1, 0, 2)


# speedup vs baseline: 1.1464x; 1.1464x over previous
import functools

import jax
import jax.numpy as jnp
from jax import lax
from jax.experimental import pallas as pl
from jax.experimental.pallas import tpu as pltpu
from jax.experimental.pallas import tpu_sc as plsc

D_MODEL = 64
SCALE = 8.0
NC, NS, L = 2, 16, 16
NW = NC * NS
CH = 128
NBUF = 4


def _make_sc_lookup(seq: int, n_rows: int):
    mesh = plsc.VectorSubcoreMesh(core_axis_name="c", subcore_axis_name="s")
    n_groups = seq // NBUF

    @functools.partial(
        pl.kernel,
        out_type=jax.ShapeDtypeStruct((seq, n_rows, D_MODEL), jnp.float32),
        mesh=mesh,
        scratch_types=[
            pltpu.VMEM((seq, CH), jnp.int32),
            [pltpu.VMEM((CH, D_MODEL), jnp.float32)] * NBUF,
            [pltpu.VMEM((CH, D_MODEL), jnp.float32)] * NBUF,
            [pltpu.SemaphoreType.DMA] * NBUF,
        ],
        compiler_params=pltpu.CompilerParams(use_tc_tiling_on_sc=False),
    )
    def k(xt_hbm, table_hbm, out_hbm, idx_v, bufs, obufs, gsems):
        wid = lax.axis_index("s") * NC + lax.axis_index("c")
        col = wid * CH
        pltpu.sync_copy(xt_hbm.at[:, pl.ds(col, CH)], idx_v)

        def g_start(t, b):
            pltpu.async_copy(table_hbm.at[idx_v.at[t]], bufs[b], gsems[b])

        def g_wait(t, b):
            pltpu.make_async_copy(
                table_hbm.at[idx_v.at[t]], bufs[b], gsems[b]
            ).wait()

        def scale(b):
            buf, obuf = bufs[b], obufs[b]

            def srow(r, c2):
                for u in range(2):
                    for c in range(D_MODEL // L):
                        sl = pl.ds(c * L, L)
                        obuf[2 * r + u, sl] = buf[2 * r + u, sl] * SCALE
                return c2

            lax.fori_loop(0, CH // 2, srow, 0)

        def s_sync(t, b):
            pltpu.sync_copy(obufs[b], out_hbm.at[t, pl.ds(col, CH)])

        for b in range(NBUF):
            g_start(b, b)

        def step(g, carry):
            for b in range(NBUF):
                t = g * NBUF + b
                g_wait(t, b)
                scale(b)
                g_start(t + NBUF, b)
                s_sync(t, b)
            return carry

        lax.fori_loop(0, n_groups - 1, step, 0)

        for b in range(NBUF):
            t = (n_groups - 1) * NBUF + b
            g_wait(t, b)
            scale(b)
            s_sync(t, b)

    return k


def kernel(x, table):
    n_rows, seq = x.shape
    outp = _make_sc_lookup(seq, n_rows)(x.T.astype(jnp.int32), table)
    return outp.transpose(1, 0, 2)
